# baseline (speedup 1.0000x reference)
import functools

import jax
import jax.numpy as jnp
from jax.experimental import pallas as pl
from jax.experimental.pallas import tpu as pltpu


_ROW_SPLIT = 1
_H_CHUNKS = 4


def _i2st_block(x_ref, m_ref, se_ref, wp_ref, bp_ref, g_ref, lb_ref,
                w1_ref, b1_ref, w2_ref, b2_ref, out_ref):
    bf = jnp.bfloat16
    wp = wp_ref[...]
    w1 = w1_ref[...]
    w2 = w2_ref[...]
    b1 = b1_ref[...]
    a = jnp.asarray(0.7978845608028654, bf)
    b = jnp.asarray(0.7978845608028654 * 0.044715, bf)
    rows = x_ref.shape[0] // _ROW_SPLIT
    ck = w1.shape[1] // _H_CHUNKS
    for r in range(_ROW_SPLIT):
        sl = pl.ds(r * rows, rows)
        proj = jnp.dot(x_ref[sl, :].astype(bf), wp,
                       preferred_element_type=jnp.float32)
        proj = proj + bp_ref[...]
        scene = jnp.where(m_ref[sl, :], proj, se_ref[sl, :])
        mu = jnp.mean(scene, axis=-1, keepdims=True)
        cen = scene - mu
        var = jnp.mean(cen * cen, axis=-1, keepdims=True)
        h = cen * jax.lax.rsqrt(var + 1e-5) * g_ref[...] + lb_ref[...]
        hb = h.astype(bf)
        gks = []
        for k in range(_H_CHUNKS):
            ffk = jnp.dot(hb, w1[:, k * ck:(k + 1) * ck],
                          preferred_element_type=jnp.float32)
            ffk = ffk.astype(bf) + b1[:, k * ck:(k + 1) * ck]
            half = jnp.asarray(0.5, bf) * ffk
            gks.append(half + half * jax.lax.erf(
                ffk * jnp.asarray(0.7071067811865476, bf)))
        ff = jnp.concatenate(gks, axis=1)
        acc = jnp.dot(ff, w2, preferred_element_type=jnp.float32)
        out_ref[sl, :] = h + acc + b2_ref[...]


@functools.partial(jax.jit, static_argnames=("block_n",))
def _i2st(x, mask, scene_embed, W_proj, b_proj, ln_g, ln_b, W1, b1, W2, b2,
          block_n=4096):
    n, c = x.shape
    h_dim = W1.shape[1]
    grid = (n // block_n,)
    row_spec = pl.BlockSpec((block_n, c), lambda i: (i, 0))
    full = lambda a: pl.BlockSpec(a.shape, lambda i: (0,) * a.ndim)
    return pl.pallas_call(
        _i2st_block,
        grid=grid,
        in_specs=[
            row_spec,
            pl.BlockSpec((block_n, 1), lambda i: (i, 0)),
            row_spec,
            full(W_proj), full(b_proj), full(ln_g), full(ln_b),
            full(W1), full(b1), full(W2), full(b2),
        ],
        out_specs=row_spec,
        out_shape=jax.ShapeDtypeStruct((n, c), jnp.float32),
    )(x, mask, scene_embed, W_proj, b_proj, ln_g, ln_b, W1, b1, W2, b2)


def kernel(x, fov_mask, scene_embed, W_proj, b_proj, ln_g, ln_b, W1, b1, W2, b2):
    b, n, c = x.shape
    h_dim = W1.shape[1]
    x2 = x.reshape(b * n, c)
    mask = fov_mask.reshape(b * n, 1)
    if b == 1:
        se = scene_embed
    else:
        se = jnp.broadcast_to(scene_embed[None], (b, n, c)).reshape(b * n, c)
    bf = jnp.bfloat16
    out = _i2st(x2, mask, se,
                W_proj.astype(bf), b_proj.reshape(1, c), ln_g.reshape(1, c),
                ln_b.reshape(1, c), W1.astype(bf), b1.reshape(1, h_dim).astype(bf),
                W2.astype(bf), b2.reshape(1, c))
    return out.reshape(b, n, c)


# speedup vs baseline: 1.5018x; 1.5018x over previous
import functools

import jax
import jax.numpy as jnp
from jax.experimental import pallas as pl
from jax.experimental.pallas import tpu as pltpu


_ROW_SPLIT = 1
_H_CHUNKS = 4


def _i2st_block(x_ref, m_ref, se_ref, wp_ref, bp_ref, g_ref, lb_ref,
                w1_ref, b1_ref, w2_ref, b2_ref, out_ref):
    bf = jnp.bfloat16
    wp = wp_ref[...]
    w1 = w1_ref[...]
    w2 = w2_ref[...]
    b1 = b1_ref[...]
    a = jnp.asarray(0.7978845608028654, bf)
    b = jnp.asarray(0.7978845608028654 * 0.044715, bf)
    rows = x_ref.shape[0] // _ROW_SPLIT
    ck = w1.shape[1] // _H_CHUNKS
    for r in range(_ROW_SPLIT):
        sl = pl.ds(r * rows, rows)
        proj = jnp.dot(x_ref[sl, :].astype(bf), wp,
                       preferred_element_type=jnp.float32)
        proj = proj + bp_ref[...]
        mb = m_ref[pl.ds(r * rows // 128, rows // 128), :].astype(bf)
        ones_row = jnp.ones((1, 128), bf)
        se_blk = se_ref[sl, :]
        parts = []
        for g in range(rows // 128):
            mexp = jax.lax.dot_general(
                mb[g:g + 1, :], ones_row,
                (((0,), (0,)), ((), ())),
                preferred_element_type=jnp.float32)
            pg = proj[g * 128:(g + 1) * 128, :]
            sg = se_blk[g * 128:(g + 1) * 128, :]
            parts.append(sg + mexp * (pg - sg))
        scene = jnp.concatenate(parts, axis=0)
        mu = jnp.mean(scene, axis=-1, keepdims=True)
        cen = scene - mu
        var = jnp.mean(cen * cen, axis=-1, keepdims=True)
        h = cen * jax.lax.rsqrt(var + 1e-5) * g_ref[...] + lb_ref[...]
        hb = h.astype(bf)
        gks = []
        for k in range(_H_CHUNKS):
            ffk = jnp.dot(hb, w1[:, k * ck:(k + 1) * ck],
                          preferred_element_type=jnp.float32)
            ffk = ffk.astype(bf) + b1[:, k * ck:(k + 1) * ck]
            half = jnp.asarray(0.5, bf) * ffk
            gks.append(half + half * jax.lax.erf(
                ffk * jnp.asarray(0.7071067811865476, bf)))
        ff = jnp.concatenate(gks, axis=1)
        acc = jnp.dot(ff, w2, preferred_element_type=jnp.float32)
        out_ref[sl, :] = h + acc + b2_ref[...]


@functools.partial(jax.jit, static_argnames=("block_n",))
def _i2st(x, mask, scene_embed, W_proj, b_proj, ln_g, ln_b, W1, b1, W2, b2,
          block_n=4096):
    n, c = x.shape
    h_dim = W1.shape[1]
    mask = mask.reshape(n // 128, 128)
    grid = (n // block_n,)
    row_spec = pl.BlockSpec((block_n, c), lambda i: (i, 0))
    full = lambda a: pl.BlockSpec(a.shape, lambda i: (0,) * a.ndim)
    return pl.pallas_call(
        _i2st_block,
        grid=grid,
        in_specs=[
            row_spec,
            pl.BlockSpec((block_n // 128, 128), lambda i: (i, 0)),
            row_spec,
            full(W_proj), full(b_proj), full(ln_g), full(ln_b),
            full(W1), full(b1), full(W2), full(b2),
        ],
        out_specs=row_spec,
        out_shape=jax.ShapeDtypeStruct((n, c), jnp.float32),
    )(x, mask, scene_embed, W_proj, b_proj, ln_g, ln_b, W1, b1, W2, b2)


def kernel(x, fov_mask, scene_embed, W_proj, b_proj, ln_g, ln_b, W1, b1, W2, b2):
    b, n, c = x.shape
    h_dim = W1.shape[1]
    x2 = x.reshape(b * n, c)
    mask = fov_mask.reshape(b * n)
    if b == 1:
        se = scene_embed
    else:
        se = jnp.broadcast_to(scene_embed[None], (b, n, c)).reshape(b * n, c)
    bf = jnp.bfloat16
    out = _i2st(x2, mask, se,
                W_proj.astype(bf), b_proj.reshape(1, c), ln_g.reshape(1, c),
                ln_b.reshape(1, c), W1.astype(bf), b1.reshape(1, h_dim).astype(bf),
                W2.astype(bf), b2.reshape(1, c))
    return out.reshape(b, n, c)


# speedup vs baseline: 1.5956x; 1.0625x over previous
import functools

import jax
import jax.numpy as jnp
from jax.experimental import pallas as pl
from jax.experimental.pallas import tpu as pltpu


_ROW_SPLIT = 1
_H_CHUNKS = 4


def _i2st_block(x_ref, m_ref, se_ref, wp_ref, bp_ref, g_ref, lb_ref,
                w1_ref, b1_ref, w2_ref, b2_ref, out_ref):
    bf = jnp.bfloat16
    wp = wp_ref[...]
    w1 = w1_ref[...]
    w2 = w2_ref[...]
    b1 = b1_ref[...]
    a = jnp.asarray(0.7978845608028654, bf)
    b = jnp.asarray(0.7978845608028654 * 0.044715, bf)
    rows = x_ref.shape[0] // _ROW_SPLIT
    ck = w1.shape[1] // _H_CHUNKS
    for r in range(_ROW_SPLIT):
        sl = pl.ds(r * rows, rows)
        proj = jnp.dot(x_ref[sl, :].astype(bf), wp,
                       preferred_element_type=jnp.float32)
        proj = proj + bp_ref[...]
        mb = m_ref[pl.ds(r * rows // 128, rows // 128), :].astype(bf)
        ones_row = jnp.ones((1, 128), bf)
        se_blk = se_ref[sl, :]
        parts = []
        for g in range(rows // 128):
            mexp = jax.lax.dot_general(
                mb[g:g + 1, :], ones_row,
                (((0,), (0,)), ((), ())),
                preferred_element_type=jnp.float32)
            pg = proj[g * 128:(g + 1) * 128, :]
            sg = se_blk[g * 128:(g + 1) * 128, :]
            parts.append(sg + mexp * (pg - sg))
        scene = jnp.concatenate(parts, axis=0)
        mu = jnp.mean(scene, axis=-1, keepdims=True)
        cen = scene - mu
        var = jnp.mean(cen * cen, axis=-1, keepdims=True)
        h = cen * jax.lax.rsqrt(var + 1e-5) * g_ref[...] + lb_ref[...]
        hb = h.astype(bf)
        gks = []
        for k in range(_H_CHUNKS):
            ffk = jnp.dot(hb, w1[:, k * ck:(k + 1) * ck],
                          preferred_element_type=jnp.float32)
            ffk = ffk.astype(bf) + b1[:, k * ck:(k + 1) * ck]
            half = jnp.asarray(0.5, bf) * ffk
            gks.append(half + half * jax.lax.erf(
                ffk * jnp.asarray(0.7071067811865476, bf)))
        ff = jnp.concatenate(gks, axis=1)
        acc = jnp.dot(ff, w2, preferred_element_type=jnp.float32)
        out_ref[sl, :] = h + acc + b2_ref[...]


@functools.partial(jax.jit, static_argnames=("block_n",))
def _i2st(x, mask, scene_embed, W_proj, b_proj, ln_g, ln_b, W1, b1, W2, b2,
          block_n=8192):
    n, c = x.shape
    h_dim = W1.shape[1]
    mask = mask.reshape(n // 128, 128)
    grid = (n // block_n,)
    row_spec = pl.BlockSpec((block_n, c), lambda i: (i, 0))
    full = lambda a: pl.BlockSpec(a.shape, lambda i: (0,) * a.ndim)
    return pl.pallas_call(
        _i2st_block,
        grid=grid,
        in_specs=[
            row_spec,
            pl.BlockSpec((block_n // 128, 128), lambda i: (i, 0)),
            row_spec,
            full(W_proj), full(b_proj), full(ln_g), full(ln_b),
            full(W1), full(b1), full(W2), full(b2),
        ],
        out_specs=row_spec,
        out_shape=jax.ShapeDtypeStruct((n, c), jnp.float32),
    )(x, mask, scene_embed, W_proj, b_proj, ln_g, ln_b, W1, b1, W2, b2)


def kernel(x, fov_mask, scene_embed, W_proj, b_proj, ln_g, ln_b, W1, b1, W2, b2):
    b, n, c = x.shape
    h_dim = W1.shape[1]
    x2 = x.reshape(b * n, c)
    mask = fov_mask.reshape(b * n)
    if b == 1:
        se = scene_embed
    else:
        se = jnp.broadcast_to(scene_embed[None], (b, n, c)).reshape(b * n, c)
    bf = jnp.bfloat16
    out = _i2st(x2, mask, se,
                W_proj.astype(bf), b_proj.reshape(1, c), ln_g.reshape(1, c),
                ln_b.reshape(1, c), W1.astype(bf), b1.reshape(1, h_dim).astype(bf),
                W2.astype(bf), b2.reshape(1, c))
    return out.reshape(b, n, c)


# speedup vs baseline: 1.5958x; 1.0002x over previous
import functools

import jax
import jax.numpy as jnp
from jax.experimental import pallas as pl
from jax.experimental.pallas import tpu as pltpu


_ROW_SPLIT = 1
_H_CHUNKS = 4


def _i2st_block(x_ref, m_ref, se_ref, wp_ref, bp_ref, g_ref, lb_ref,
                w1_ref, b1_ref, w2_ref, b2_ref, out_ref):
    bf = jnp.bfloat16
    wp = wp_ref[...]
    w1 = w1_ref[...]
    w2 = w2_ref[...]
    b1 = b1_ref[...]
    rows = x_ref.shape[0] // _ROW_SPLIT
    ck = w1.shape[1] // _H_CHUNKS
    for r in range(_ROW_SPLIT):
        sl = pl.ds(r * rows, rows)
        proj = jnp.dot(x_ref[sl, :].astype(bf), wp,
                       preferred_element_type=jnp.float32)
        proj = proj + bp_ref[...]
        mb = m_ref[pl.ds(r * rows // 128, rows // 128), :].astype(bf)
        ones_row = jnp.ones((1, 128), bf)
        se_blk = se_ref[sl, :]
        parts = []
        for g in range(rows // 128):
            mexp = jax.lax.dot_general(
                mb[g:g + 1, :], ones_row,
                (((0,), (0,)), ((), ())),
                preferred_element_type=jnp.float32)
            pg = proj[g * 128:(g + 1) * 128, :]
            sg = se_blk[g * 128:(g + 1) * 128, :]
            parts.append(sg + mexp * (pg - sg))
        scene = jnp.concatenate(parts, axis=0)
        mu = jnp.mean(scene, axis=-1, keepdims=True)
        cen = scene - mu
        var = jnp.mean(cen * cen, axis=-1, keepdims=True)
        h = cen * jax.lax.rsqrt(var + 1e-5) * g_ref[...] + lb_ref[...]
        hb = h.astype(bf)
        gks = []
        for k in range(_H_CHUNKS):
            ffk = jnp.dot(hb, w1[:, k * ck:(k + 1) * ck],
                          preferred_element_type=jnp.float32)
            ffk = ffk.astype(bf) + b1[:, k * ck:(k + 1) * ck]
            half = jnp.asarray(0.5, bf) * ffk
            gks.append(half + half * jax.lax.erf(
                ffk * jnp.asarray(0.7071067811865476, bf)))
        ff = jnp.concatenate(gks, axis=1)
        acc = jnp.dot(ff, w2, preferred_element_type=jnp.float32)
        out_ref[sl, :] = h + acc + b2_ref[...]


@functools.partial(jax.jit, static_argnames=("block_n",))
def _i2st(x, mask, scene_embed, W_proj, b_proj, ln_g, ln_b, W1, b1, W2, b2,
          block_n=8192):
    n, c = x.shape
    h_dim = W1.shape[1]
    mask = mask.reshape(n // 128, 128)
    grid = (n // block_n,)
    row_spec = pl.BlockSpec((block_n, c), lambda i: (i, 0))
    full = lambda a: pl.BlockSpec(a.shape, lambda i: (0,) * a.ndim)
    return pl.pallas_call(
        _i2st_block,
        grid=grid,
        in_specs=[
            row_spec,
            pl.BlockSpec((block_n // 128, 128), lambda i: (i, 0)),
            row_spec,
            full(W_proj), full(b_proj), full(ln_g), full(ln_b),
            full(W1), full(b1), full(W2), full(b2),
        ],
        out_specs=row_spec,
        out_shape=jax.ShapeDtypeStruct((n, c), jnp.float32),
    )(x, mask, scene_embed, W_proj, b_proj, ln_g, ln_b, W1, b1, W2, b2)


def kernel(x, fov_mask, scene_embed, W_proj, b_proj, ln_g, ln_b, W1, b1, W2, b2):
    b, n, c = x.shape
    h_dim = W1.shape[1]
    x2 = x.reshape(b * n, c)
    mask = fov_mask.reshape(b * n)
    if b == 1:
        se = scene_embed
    else:
        se = jnp.broadcast_to(scene_embed[None], (b, n, c)).reshape(b * n, c)
    bf = jnp.bfloat16
    out = _i2st(x2, mask, se,
                W_proj.astype(bf), b_proj.reshape(1, c), ln_g.reshape(1, c),
                ln_b.reshape(1, c), W1.astype(bf), b1.reshape(1, h_dim).astype(bf),
                W2.astype(bf), b2.reshape(1, c))
    return out.reshape(b, n, c)
